# Initial kernel scaffold; baseline (speedup 1.0000x reference)
import jax
import jax.numpy as jnp
from jax.experimental import pallas as pl


def kernel(x, tensor_pool, W1, b1, W2, b2, temperature, Wp, bp, gamma, beta, Wm, bm):
    raise NotImplementedError("write your pallas kernel here")


# speedup vs baseline: 2.9402x; 2.9402x over previous
import functools
import math

import jax
import jax.numpy as jnp
from jax.experimental import pallas as pl
from jax.experimental.pallas import tpu as pltpu

_NPOOL = 64
_TOPK = 2
_DSCALE = 0.01


def _router_block(x_ref, W1_ref, b1_ref, W2_ref, b2_ref, temp_ref, Wp_ref,
                  bp_ref, gamma_ref, beta_ref, Wm_ref, bm_ref, pool_ref,
                  out_ref, loss_ref, usage_acc, *, scale):
    i = pl.program_id(0)
    n = pl.num_programs(0)
    xb = x_ref[...]

    inter = jax.lax.dot_general(
        xb, W1_ref[...], (((1,), (0,)), ((), ())),
        precision=jax.lax.Precision.DEFAULT,
        preferred_element_type=jnp.float32)
    inter = jnp.maximum(inter + b1_ref[...], 0.0)
    logits = jax.lax.dot_general(
        inter, W2_ref[...], (((1,), (0,)), ((), ())),
        precision=jax.lax.Precision.DEFAULT,
        preferred_element_type=jnp.float32) + b2_ref[...]

    temp = jnp.clip(temp_ref[0, 0], 0.1, 5.0)
    scaled = jnp.clip(logits / temp, -10.0, 10.0)

    col = jax.lax.broadcasted_iota(jnp.int32, scaled.shape, 1)
    m1 = jnp.max(scaled, axis=-1, keepdims=True)
    idx1 = jnp.min(jnp.where(scaled == m1, col, _NPOOL), axis=-1,
                   keepdims=True)
    masked = jnp.where(col == idx1, -jnp.inf, scaled)
    m2 = jnp.max(masked, axis=-1, keepdims=True)
    idx2 = jnp.min(jnp.where(masked == m2, col, _NPOOL), axis=-1,
                   keepdims=True)
    e2 = jnp.exp(m2 - m1)
    denom = 1.0 + e2
    w1 = 1.0 / denom
    w2 = e2 / denom
    wmat = (jnp.where(col == idx1, w1, 0.0)
            + jnp.where(col == idx2, w2, 0.0))

    @pl.when(i == 0)
    def _():
        usage_acc[...] = jnp.zeros_like(usage_acc)
    usage_acc[...] += jnp.sum(wmat, axis=0, keepdims=True)

    wmap = jax.lax.dot_general(
        wmat, pool_ref[...], (((1,), (0,)), ((), ())),
        precision=jax.lax.Precision.HIGHEST,
        preferred_element_type=jnp.float32)

    px = jax.lax.dot_general(
        xb.astype(jnp.bfloat16), Wp_ref[...], (((1,), (0,)), ((), ())),
        preferred_element_type=jnp.float32) + bp_ref[...]
    mu = jnp.mean(px, axis=-1, keepdims=True)
    var = jnp.mean((px - mu) ** 2, axis=-1, keepdims=True)
    ln = ((px - mu) / jnp.sqrt(var + 1e-5)) * gamma_ref[...] + beta_ref[...]

    comb = jnp.concatenate([ln, wmap], axis=-1).astype(jnp.bfloat16)
    out = jax.lax.dot_general(
        comb, Wm_ref[...], (((1,), (0,)), ((), ())),
        preferred_element_type=jnp.float32) + bm_ref[...]
    out_ref[...] = out

    @pl.when(i == n - 1)
    def _():
        u = usage_acc[...]
        uf = u / (jnp.sum(u) + 1e-8)
        d = uf - 1.0 / _NPOOL
        loss_ref[...] = (jnp.mean(d * d) * (scale * _DSCALE)).reshape(1, 1)


def kernel(x, tensor_pool, W1, b1, W2, b2, temperature, Wp, bp, gamma, beta,
           Wm, bm):
    B, S, H = x.shape
    M = B * S
    npool, tdim = tensor_pool.shape
    inter_dim = W1.shape[1]
    BM = 512
    grid = (M // BM,)
    scale = min(1.0, float(x.size) / (npool * _TOPK))

    xf = x.reshape(M, H)
    full = lambda shape: pl.BlockSpec(shape, lambda i: (0,) * len(shape))
    out, loss = pl.pallas_call(
        functools.partial(_router_block, scale=scale),
        grid=grid,
        in_specs=[
            pl.BlockSpec((BM, H), lambda i: (i, 0)),
            full((H, inter_dim)),
            full((1, inter_dim)),
            full((inter_dim, npool)),
            full((1, npool)),
            full((1, 1)),
            full((H, tdim)),
            full((1, tdim)),
            full((1, tdim)),
            full((1, tdim)),
            full((2 * tdim, tdim)),
            full((1, tdim)),
            full((npool, tdim)),
        ],
        out_specs=(
            pl.BlockSpec((BM, tdim), lambda i: (i, 0)),
            pl.BlockSpec((1, 1), lambda i: (0, 0)),
        ),
        out_shape=(
            jax.ShapeDtypeStruct((M, tdim), jnp.float32),
            jax.ShapeDtypeStruct((1, 1), jnp.float32),
        ),
        scratch_shapes=[pltpu.VMEM((1, npool), jnp.float32)],
    )(xf, W1, b1.reshape(1, -1), W2, b2.reshape(1, -1),
      temperature.reshape(1, 1), Wp.astype(jnp.bfloat16), bp.reshape(1, -1),
      gamma.reshape(1, -1), beta.reshape(1, -1), Wm.astype(jnp.bfloat16),
      bm.reshape(1, -1), tensor_pool)
    return out.reshape(B, S, tdim), loss[0, 0]
